# baseline (speedup 1.0000x reference)
import functools

import jax
import jax.numpy as jnp
from jax import lax
from jax.experimental import pallas as pl
from jax.experimental.pallas import tpu as pltpu
from jax.experimental.pallas import tpu_sc as plsc

B = 4096
L = 200
D = 64
V = 1000000
R = 8192
G2 = 61
CUT1 = G2 * R
CUT2 = 2 * CUT1
TAIL = V - CUT2
TH = TAIL // 2
H3 = (G2 + 1) * R
V2 = 2 * H3
NC = 2
NS = 16
NW = NC * NS
RPW = B // NW
LA = 128
LB = L - LA
LANES = 16
KV = D // LANES
UNROLL = 8


def _repack_table(table):
  tt = jnp.swapaxes(table, 0, 1)
  tail = lax.dynamic_slice(tt, (0, CUT2), (D, TAIL))

  def body(a_ref, b_ref, t_ref, o_ref):
    c = pl.program_id(0)

    @pl.when(c < G2)
    def _():
      ta = jnp.transpose(a_ref[...], (1, 0))
      eye = jnp.eye(D, dtype=jnp.float32)
      tb = lax.dot_general(
          b_ref[...], eye, (((0,), (0,)), ((), ())),
          preferred_element_type=jnp.float32)
      o_ref[...] = jnp.concatenate([ta, tb], axis=1)

    @pl.when(c == G2)
    def _():
      o_ref[0:TH, 0:D] = jnp.transpose(t_ref[:, 0:TH], (1, 0))
      o_ref[0:TH, D:2 * D] = jnp.transpose(t_ref[:, TH:TAIL], (1, 0))

  packed = pl.pallas_call(
      body,
      grid=(G2 + 1,),
      compiler_params=pltpu.CompilerParams(
          dimension_semantics=("parallel",)),
      in_specs=[
          pl.BlockSpec((D, R), lambda c: (0, jnp.minimum(c, G2 - 1))),
          pl.BlockSpec((D, R), lambda c: (0, jnp.minimum(c + G2, 2 * G2 - 1))),
          pl.BlockSpec((D, TAIL), lambda c: (0, 0)),
      ],
      out_specs=pl.BlockSpec((R, 2 * D), lambda c: (c, 0)),
      out_shape=jax.ShapeDtypeStruct((H3, 2 * D), jnp.float32),
  )(tt, tt, tail)
  return packed.reshape(V2, D)


def _pooled_lookup(items, table_p):
  mesh = plsc.VectorSubcoreMesh(core_axis_name="c", subcore_axis_name="s")

  @functools.partial(
      pl.kernel,
      mesh=mesh,
      compiler_params=pltpu.CompilerParams(use_tc_tiling_on_sc=False),
      out_type=jax.ShapeDtypeStruct((B, D), jnp.float32),
      scratch_types=[
          pltpu.VMEM((RPW, LA), jnp.int32),
          pltpu.VMEM((RPW, LB), jnp.int32),
          pltpu.VMEM((2, L, D), jnp.float32),
          pltpu.VMEM((RPW, D), jnp.float32),
          pltpu.SemaphoreType.DMA,
          pltpu.SemaphoreType.DMA,
      ],
  )
  def k(items_hbm, table_hbm, out_hbm,
        idx_a, idx_b, rows_v, out_v, sem0, sem1):
    wid = lax.axis_index("s") * NC + lax.axis_index("c")
    row0 = wid * RPW
    sems = (sem0, sem1)

    pltpu.sync_copy(items_hbm.at[pl.ds(row0, RPW), pl.ds(0, LA)], idx_a)
    pltpu.sync_copy(items_hbm.at[pl.ds(row0, RPW), pl.ds(LA, LB)], idx_b)

    def gather_a(j, buf):
      return pltpu.make_async_copy(
          table_hbm.at[idx_a.at[j]],
          rows_v.at[buf, pl.ds(0, LA)],
          sems[buf])

    def gather_b(j, buf):
      return pltpu.make_async_copy(
          table_hbm.at[idx_b.at[j]],
          rows_v.at[buf, pl.ds(LA, LB)],
          sems[buf])

    def issue(j, buf):
      gather_a(j, buf).start()
      gather_b(j, buf).start()

    def wait(j, buf):
      gather_a(j, buf).wait()
      gather_b(j, buf).wait()

    issue(0, 0)
    issue(1, 1)

    scale = jnp.float32(1.0 / L)

    def accumulate(buf):
      def step(i, accs):
        for u in range(UNROLL):
          r = i * UNROLL + u
          accs = tuple(
              accs[kk] + rows_v[buf, r, pl.ds(kk * LANES, LANES)]
              for kk in range(KV))
        return accs
      init = tuple(jnp.zeros((LANES,), jnp.float32) for _ in range(KV))
      return lax.fori_loop(0, L // UNROLL, step, init)

    def outer(i, carry):
      for b2 in range(2):
        j = i * 2 + b2
        wait(j, b2)
        accs = accumulate(b2)
        nj = j + 2

        @pl.when(nj < RPW)
        def _():
          issue(nj, b2)

        for kk in range(KV):
          out_v[j, pl.ds(kk * LANES, LANES)] = accs[kk] * scale
      return carry

    lax.fori_loop(0, RPW // 2, outer, 0)
    pltpu.sync_copy(out_v, out_hbm.at[pl.ds(row0, RPW)])

  return k(items, table_p)


def kernel(items, table):
  table_p = _repack_table(table)
  it = items.astype(jnp.int32)
  rows = jnp.where(
      it < CUT1, 2 * it,
      jnp.where(
          it < CUT2, 2 * it - (2 * CUT1 - 1),
          jnp.where(it < CUT2 + TH, 2 * it - CUT2, 2 * it - (CUT2 + TAIL - 1))))
  return _pooled_lookup(rows, table_p)


# speedup vs baseline: 1.1211x; 1.1211x over previous
import functools

import jax
import jax.numpy as jnp
from jax import lax
from jax.experimental import pallas as pl
from jax.experimental.pallas import tpu as pltpu
from jax.experimental.pallas import tpu_sc as plsc

B = 4096
L = 200
D = 64
V = 1000000
R = 8192
G2 = 61
CUT1 = G2 * R
CUT2 = 2 * CUT1
TAIL = V - CUT2
TH = TAIL // 2
H3 = (G2 + 1) * R
V2 = 2 * H3
NC = 2
NS = 16
NW = NC * NS
RPW = B // NW
LA = 128
LB = L - LA
LANES = 16
KV = D // LANES
UNROLL = 8


def _repack_table(table):
  tt = jnp.swapaxes(table, 0, 1)
  tail = lax.dynamic_slice(tt, (0, CUT2), (D, TAIL))

  def body(a_ref, b_ref, t_ref, o_ref):
    c = pl.program_id(0)

    @pl.when(c < G2)
    def _():
      o_ref[...] = jnp.concatenate(
          [jnp.transpose(a_ref[...], (1, 0)),
           jnp.transpose(b_ref[...], (1, 0))], axis=1)

    @pl.when(c == G2)
    def _():
      o_ref[0:TH, 0:D] = jnp.transpose(t_ref[:, 0:TH], (1, 0))
      o_ref[0:TH, D:2 * D] = jnp.transpose(t_ref[:, TH:TAIL], (1, 0))

  packed = pl.pallas_call(
      body,
      grid=(G2 + 1,),
      compiler_params=pltpu.CompilerParams(
          dimension_semantics=("parallel",)),
      in_specs=[
          pl.BlockSpec((D, R), lambda c: (0, jnp.minimum(c, G2 - 1))),
          pl.BlockSpec((D, R), lambda c: (0, jnp.minimum(c + G2, 2 * G2 - 1))),
          pl.BlockSpec((D, TAIL), lambda c: (0, 0)),
      ],
      out_specs=pl.BlockSpec((R, 2 * D), lambda c: (c, 0)),
      out_shape=jax.ShapeDtypeStruct((H3, 2 * D), jnp.float32),
  )(tt, tt, tail)
  return packed.reshape(V2, D)


def _pooled_lookup(items, table_p):
  mesh = plsc.VectorSubcoreMesh(core_axis_name="c", subcore_axis_name="s")

  @functools.partial(
      pl.kernel,
      mesh=mesh,
      compiler_params=pltpu.CompilerParams(use_tc_tiling_on_sc=False),
      out_type=jax.ShapeDtypeStruct((B, D), jnp.float32),
      scratch_types=[
          pltpu.VMEM((RPW, LA), jnp.int32),
          pltpu.VMEM((RPW, LB), jnp.int32),
          pltpu.VMEM((4, L, D), jnp.float32),
          pltpu.VMEM((RPW, D), jnp.float32),
          pltpu.SemaphoreType.DMA,
          pltpu.SemaphoreType.DMA,
          pltpu.SemaphoreType.DMA,
          pltpu.SemaphoreType.DMA,
      ],
  )
  def k(items_hbm, table_hbm, out_hbm,
        idx_a, idx_b, rows_v, out_v, sem0, sem1, sem2, sem3):
    wid = lax.axis_index("s") * NC + lax.axis_index("c")
    row0 = wid * RPW
    sems = (sem0, sem1, sem2, sem3)

    pltpu.sync_copy(items_hbm.at[pl.ds(row0, RPW), pl.ds(0, LA)], idx_a)
    pltpu.sync_copy(items_hbm.at[pl.ds(row0, RPW), pl.ds(LA, LB)], idx_b)

    def gather_a(j, buf):
      return pltpu.make_async_copy(
          table_hbm.at[idx_a.at[j]],
          rows_v.at[buf, pl.ds(0, LA)],
          sems[buf])

    def gather_b(j, buf):
      return pltpu.make_async_copy(
          table_hbm.at[idx_b.at[j]],
          rows_v.at[buf, pl.ds(LA, LB)],
          sems[buf])

    def issue(j, buf):
      gather_a(j, buf).start()
      gather_b(j, buf).start()

    def wait(j, buf):
      gather_a(j, buf).wait()
      gather_b(j, buf).wait()

    issue(0, 0)
    issue(1, 1)
    issue(2, 2)
    issue(3, 3)

    scale = jnp.float32(1.0 / L)

    def accumulate(buf):
      def step(i, accs):
        for u in range(UNROLL):
          r = i * UNROLL + u
          accs = tuple(
              accs[kk] + rows_v[buf, r, pl.ds(kk * LANES, LANES)]
              for kk in range(KV))
        return accs
      init = tuple(jnp.zeros((LANES,), jnp.float32) for _ in range(KV))
      return lax.fori_loop(0, L // UNROLL, step, init)

    def outer(i, carry):
      for b2 in range(4):
        j = i * 4 + b2
        wait(j, b2)
        accs = accumulate(b2)
        nj = j + 4

        @pl.when(nj < RPW)
        def _():
          issue(nj, b2)

        for kk in range(KV):
          out_v[j, pl.ds(kk * LANES, LANES)] = accs[kk] * scale
      return carry

    lax.fori_loop(0, RPW // 4, outer, 0)
    pltpu.sync_copy(out_v, out_hbm.at[pl.ds(row0, RPW)])

  return k(items, table_p)


def kernel(items, table):
  table_p = _repack_table(table)
  it = items.astype(jnp.int32)
  rows = jnp.where(
      it < CUT1, 2 * it,
      jnp.where(
          it < CUT2, 2 * it - (2 * CUT1 - 1),
          jnp.where(it < CUT2 + TH, 2 * it - CUT2, 2 * it - (CUT2 + TAIL - 1))))
  return _pooled_lookup(rows, table_p)
